# baseline (speedup 1.0000x reference)
import functools

import jax
import jax.numpy as jnp
import numpy as np
from jax.experimental import pallas as pl

IN_CH = 256
HID = 128
N_HEAD = 8
D_K = HID // N_HEAD
N = 2048
BLK = 256
GRID = N // BLK


def _proj_kernel(z_ref, pe_ref, wh_ref, bh_ref, wpe_ref, bpe_ref,
                 wq_ref, wk_ref, wv_ref, h_ref, q_ref, k_ref, v_ref):
    h = (jnp.dot(z_ref[...], wh_ref[...], preferred_element_type=jnp.float32)
         + bh_ref[...]
         + jnp.dot(pe_ref[...], wpe_ref[...], preferred_element_type=jnp.float32)
         + bpe_ref[...])
    h_ref[...] = h
    q_ref[...] = jnp.dot(h, wq_ref[...], preferred_element_type=jnp.float32)
    k_ref[...] = jnp.dot(h, wk_ref[...], preferred_element_type=jnp.float32)
    v_ref[...] = jnp.dot(h, wv_ref[...], preferred_element_type=jnp.float32)


def _attn_ffn_kernel(adj_ref, q_ref, k_ref, v_ref, h_ref,
                     wo_ref, bo_ref, w1_ref, b1_ref, w2_ref, b2_ref,
                     g1_ref, bb1_ref, g2_ref, bb2_ref, out_ref):
    j = pl.program_id(0)
    mask = (adj_ref[...] == 1).astype(jnp.float32)
    srow = jax.lax.broadcasted_iota(jnp.int32, (N, BLK), 0)
    dcol = jax.lax.broadcasted_iota(jnp.int32, (N, BLK), 1) + j * BLK
    mask = mask + (srow == dcol).astype(jnp.float32)

    cols = []
    for hh in range(N_HEAD):
        sl = slice(hh * D_K, (hh + 1) * D_K)
        kh = k_ref[:, sl]
        qh = q_ref[:, sl]
        vh = v_ref[:, sl]
        s = jax.lax.dot_general(kh, qh, (((1,), (1,)), ((), ())),
                                preferred_element_type=jnp.float32)
        s = s / np.float32(np.sqrt(D_K))
        w = mask * jnp.exp(jnp.clip(s, -5.0, 5.0))
        wv = jax.lax.dot_general(w, vh, (((0,), (0,)), ((), ())),
                                 preferred_element_type=jnp.float32)
        zden = jnp.sum(w, axis=0)
        cols.append(wv / (zden[:, None] + 1e-6))
    attn = jnp.concatenate(cols, axis=1)

    h1 = h_ref[...] + jnp.dot(attn, wo_ref[...],
                              preferred_element_type=jnp.float32) + bo_ref[...]
    h1 = h1 / np.float32(np.sqrt(1.0 + 1e-5)) * g1_ref[...] + bb1_ref[...]
    t = jnp.maximum(jnp.dot(h1, w1_ref[...],
                            preferred_element_type=jnp.float32) + b1_ref[...], 0.0)
    h2 = h1 + jnp.dot(t, w2_ref[...],
                      preferred_element_type=jnp.float32) + b2_ref[...]
    out_ref[...] = h2 / np.float32(np.sqrt(1.0 + 1e-5)) * g2_ref[...] + bb2_ref[...]


def _indeg_kernel(adj_ref, out_ref):
    a = (adj_ref[...] == 1).astype(jnp.float32)
    out_ref[...] = jnp.sum(a, axis=0, keepdims=True) + 1.0


def _ls_kernel(adj_r_ref, adj_c_ref, ninv_row_ref, ninv_row_i_ref,
               ninv_col_ref, ninv_col_i_ref, out_ref):
    i = pl.program_id(0)
    row_g = jax.lax.broadcasted_iota(jnp.int32, (BLK, N), 0) + i * BLK
    col_g = jax.lax.broadcasted_iota(jnp.int32, (BLK, N), 1)
    eye_r = (row_g == col_g).astype(jnp.float32)
    a_r = (adj_r_ref[...] == 1).astype(jnp.float32) + eye_r
    t1 = eye_r - (ninv_col_i_ref[...] * a_r) * ninv_row_ref[...]

    srow = jax.lax.broadcasted_iota(jnp.int32, (N, BLK), 0)
    dcol = jax.lax.broadcasted_iota(jnp.int32, (N, BLK), 1) + i * BLK
    eye_c = (srow == dcol).astype(jnp.float32)
    a_c = (adj_c_ref[...] == 1).astype(jnp.float32) + eye_c
    t2 = eye_c - (ninv_col_ref[...] * a_c) * ninv_row_i_ref[...]
    out_ref[...] = 0.5 * (t1 + jnp.transpose(t2))


def _row(x):
    return x.reshape(1, -1)


@functools.partial(jax.jit, static_argnames=())
def kernel(lncrna_x, disease_x, adj, W_h, b_h, W_pe, b_pe, WQ, WK, WV,
           WO, bO, W1, b1, W2, b2, bn1_g, bn1_b, bn2_g, bn2_b):
    n = lncrna_x.shape[0] + disease_x.shape[0]
    z = jnp.concatenate([lncrna_x, disease_x], axis=0)

    indeg = pl.pallas_call(
        _indeg_kernel,
        grid=(GRID,),
        in_specs=[pl.BlockSpec((N, BLK), lambda i: (0, i))],
        out_specs=pl.BlockSpec((1, BLK), lambda i: (0, i)),
        out_shape=jax.ShapeDtypeStruct((1, N), jnp.float32),
    )(adj)
    ninv = jnp.clip(indeg.reshape(n), 1.0, None) ** -0.5
    ninv_row = ninv.reshape(1, n)
    ninv_col = ninv.reshape(n, 1)
    Ls = pl.pallas_call(
        _ls_kernel,
        grid=(GRID,),
        in_specs=[pl.BlockSpec((BLK, N), lambda i: (i, 0)),
                  pl.BlockSpec((N, BLK), lambda i: (0, i)),
                  pl.BlockSpec((1, N), lambda i: (0, 0)),
                  pl.BlockSpec((1, BLK), lambda i: (0, i)),
                  pl.BlockSpec((N, 1), lambda i: (0, 0)),
                  pl.BlockSpec((BLK, 1), lambda i: (i, 0))],
        out_specs=pl.BlockSpec((BLK, N), lambda i: (i, 0)),
        out_shape=jax.ShapeDtypeStruct((N, N), jnp.float32),
    )(adj, adj, ninv_row, ninv_row, ninv_col, ninv_col)
    _, evecs = jnp.linalg.eigh(Ls)
    pos_enc = evecs[:, 1:IN_CH + 1]

    full = lambda shape: pl.BlockSpec(shape, lambda i: (0, 0))
    rowblk = lambda w: pl.BlockSpec((BLK, w), lambda i: (i, 0))

    h, Q, K, V = pl.pallas_call(
        _proj_kernel,
        grid=(GRID,),
        in_specs=[rowblk(IN_CH), rowblk(IN_CH),
                  full((IN_CH, HID)), full((1, HID)),
                  full((IN_CH, HID)), full((1, HID)),
                  full((HID, HID)), full((HID, HID)), full((HID, HID))],
        out_specs=[rowblk(HID)] * 4,
        out_shape=[jax.ShapeDtypeStruct((N, HID), jnp.float32)] * 4,
    )(z, pos_enc, W_h, _row(b_h), W_pe, _row(b_pe), WQ, WK, WV)

    out = pl.pallas_call(
        _attn_ffn_kernel,
        grid=(GRID,),
        in_specs=[pl.BlockSpec((N, BLK), lambda i: (0, i)),
                  rowblk(HID),
                  full((N, HID)), full((N, HID)),
                  rowblk(HID),
                  full((HID, HID)), full((1, HID)),
                  full((HID, 2 * HID)), full((1, 2 * HID)),
                  full((2 * HID, HID)), full((1, HID)),
                  full((1, HID)), full((1, HID)),
                  full((1, HID)), full((1, HID))],
        out_specs=rowblk(HID),
        out_shape=jax.ShapeDtypeStruct((N, HID), jnp.float32),
    )(adj, Q, K, V, h, WO, _row(bO), W1, _row(b1), W2, _row(b2),
      _row(bn1_g), _row(bn1_b), _row(bn2_g), _row(bn2_b))
    return out


# speedup vs baseline: 1.0143x; 1.0143x over previous
import functools

import jax
import jax.numpy as jnp
import numpy as np
from jax.experimental import pallas as pl
from jax.experimental.pallas import tpu as pltpu
from jax.experimental.pallas import tpu_sc as plsc

IN_CH = 256
HID = 128
N_HEAD = 8
D_K = HID // N_HEAD
N = 2048
BLK = 256
GRID = N // BLK


def _proj_kernel(z_ref, pe_ref, wh_ref, bh_ref, wpe_ref, bpe_ref,
                 wq_ref, wk_ref, wv_ref, h_ref, q_ref, k_ref, v_ref):
    h = (jnp.dot(z_ref[...], wh_ref[...], preferred_element_type=jnp.float32)
         + bh_ref[...]
         + jnp.dot(pe_ref[...], wpe_ref[...], preferred_element_type=jnp.float32)
         + bpe_ref[...])
    h_ref[...] = h
    q_ref[...] = jnp.dot(h, wq_ref[...], preferred_element_type=jnp.float32)
    k_ref[...] = jnp.dot(h, wk_ref[...], preferred_element_type=jnp.float32)
    v_ref[...] = jnp.dot(h, wv_ref[...], preferred_element_type=jnp.float32)


def _attn_ffn_kernel(adj_ref, q_ref, k_ref, v_ref, h_ref,
                     wo_ref, bo_ref, w1_ref, b1_ref, w2_ref, b2_ref,
                     g1_ref, bb1_ref, g2_ref, bb2_ref, out_ref):
    j = pl.program_id(0)
    mask = (adj_ref[...] == 1).astype(jnp.float32)
    srow = jax.lax.broadcasted_iota(jnp.int32, (N, BLK), 0)
    dcol = jax.lax.broadcasted_iota(jnp.int32, (N, BLK), 1) + j * BLK
    mask = mask + (srow == dcol).astype(jnp.float32)

    cols = []
    for hh in range(N_HEAD):
        sl = slice(hh * D_K, (hh + 1) * D_K)
        kh = k_ref[:, sl]
        qh = q_ref[:, sl]
        vh = v_ref[:, sl]
        s = jax.lax.dot_general(kh, qh, (((1,), (1,)), ((), ())),
                                preferred_element_type=jnp.float32)
        s = s / np.float32(np.sqrt(D_K))
        w = mask * jnp.exp(jnp.clip(s, -5.0, 5.0))
        wv = jax.lax.dot_general(w, vh, (((0,), (0,)), ((), ())),
                                 preferred_element_type=jnp.float32)
        zden = jnp.sum(w, axis=0)
        cols.append(wv / (zden[:, None] + 1e-6))
    attn = jnp.concatenate(cols, axis=1)

    h1 = h_ref[...] + jnp.dot(attn, wo_ref[...],
                              preferred_element_type=jnp.float32) + bo_ref[...]
    h1 = h1 / np.float32(np.sqrt(1.0 + 1e-5)) * g1_ref[...] + bb1_ref[...]
    t = jnp.maximum(jnp.dot(h1, w1_ref[...],
                            preferred_element_type=jnp.float32) + b1_ref[...], 0.0)
    h2 = h1 + jnp.dot(t, w2_ref[...],
                      preferred_element_type=jnp.float32) + b2_ref[...]
    out_ref[...] = h2 / np.float32(np.sqrt(1.0 + 1e-5)) * g2_ref[...] + bb2_ref[...]


_SC_TILES = 32
_SC_COLS = N // _SC_TILES
_SC_ROWCHUNK = 128


def _indeg_sc_kernel(adj_hbm, out_hbm, blk_v, acc_v):
    wid = jax.lax.axis_index("s") * 2 + jax.lax.axis_index("c")
    d0 = wid * _SC_COLS
    stripe0 = (wid // 2) * 128
    half = (wid % 2) * _SC_COLS
    for g in range(_SC_COLS // 16):
        acc_v[pl.ds(g * 16, 16)] = jnp.full((16,), 1.0, jnp.float32)

    def chunk(i, _):
        pltpu.sync_copy(adj_hbm.at[pl.ds(i * _SC_ROWCHUNK, _SC_ROWCHUNK),
                                   pl.ds(stripe0, 128)], blk_v)

        def row(r, _):
            for g in range(_SC_COLS // 16):
                v = blk_v[r, pl.ds(half + g * 16, 16)]
                acc = acc_v[pl.ds(g * 16, 16)]
                acc_v[pl.ds(g * 16, 16)] = acc + jnp.where(
                    v == 1, jnp.float32(1.0), jnp.float32(0.0))
            return 0

        jax.lax.fori_loop(0, _SC_ROWCHUNK, row, 0)
        return 0

    jax.lax.fori_loop(0, N // _SC_ROWCHUNK, chunk, 0)
    pltpu.sync_copy(acc_v, out_hbm.at[pl.ds(d0, _SC_COLS)])


def _ls_kernel(adj_r_ref, adj_c_ref, ninv_row_ref, ninv_row_i_ref,
               ninv_col_ref, ninv_col_i_ref, out_ref):
    i = pl.program_id(0)
    row_g = jax.lax.broadcasted_iota(jnp.int32, (BLK, N), 0) + i * BLK
    col_g = jax.lax.broadcasted_iota(jnp.int32, (BLK, N), 1)
    eye_r = (row_g == col_g).astype(jnp.float32)
    a_r = (adj_r_ref[...] == 1).astype(jnp.float32) + eye_r
    t1 = eye_r - (ninv_col_i_ref[...] * a_r) * ninv_row_ref[...]

    srow = jax.lax.broadcasted_iota(jnp.int32, (N, BLK), 0)
    dcol = jax.lax.broadcasted_iota(jnp.int32, (N, BLK), 1) + i * BLK
    eye_c = (srow == dcol).astype(jnp.float32)
    a_c = (adj_c_ref[...] == 1).astype(jnp.float32) + eye_c
    t2 = eye_c - (ninv_col_ref[...] * a_c) * ninv_row_i_ref[...]
    out_ref[...] = 0.5 * (t1 + jnp.transpose(t2))


def _row(x):
    return x.reshape(1, -1)


@functools.partial(jax.jit, static_argnames=())
def kernel(lncrna_x, disease_x, adj, W_h, b_h, W_pe, b_pe, WQ, WK, WV,
           WO, bO, W1, b1, W2, b2, bn1_g, bn1_b, bn2_g, bn2_b):
    n = lncrna_x.shape[0] + disease_x.shape[0]
    z = jnp.concatenate([lncrna_x, disease_x], axis=0)

    indeg = pl.kernel(
        _indeg_sc_kernel,
        out_type=jax.ShapeDtypeStruct((N,), jnp.float32),
        mesh=plsc.VectorSubcoreMesh(core_axis_name="c", subcore_axis_name="s"),
        scratch_types=[pltpu.VMEM((_SC_ROWCHUNK, 128), jnp.int32),
                       pltpu.VMEM((_SC_COLS,), jnp.float32)],
    )(adj)
    ninv = jnp.clip(indeg.reshape(n), 1.0, None) ** -0.5
    ninv_row = ninv.reshape(1, n)
    ninv_col = ninv.reshape(n, 1)
    Ls = pl.pallas_call(
        _ls_kernel,
        grid=(GRID,),
        in_specs=[pl.BlockSpec((BLK, N), lambda i: (i, 0)),
                  pl.BlockSpec((N, BLK), lambda i: (0, i)),
                  pl.BlockSpec((1, N), lambda i: (0, 0)),
                  pl.BlockSpec((1, BLK), lambda i: (0, i)),
                  pl.BlockSpec((N, 1), lambda i: (0, 0)),
                  pl.BlockSpec((BLK, 1), lambda i: (i, 0))],
        out_specs=pl.BlockSpec((BLK, N), lambda i: (i, 0)),
        out_shape=jax.ShapeDtypeStruct((N, N), jnp.float32),
    )(adj, adj, ninv_row, ninv_row, ninv_col, ninv_col)
    _, evecs = jnp.linalg.eigh(Ls)
    pos_enc = evecs[:, 1:IN_CH + 1]

    full = lambda shape: pl.BlockSpec(shape, lambda i: (0, 0))
    rowblk = lambda w: pl.BlockSpec((BLK, w), lambda i: (i, 0))

    h, Q, K, V = pl.pallas_call(
        _proj_kernel,
        grid=(GRID,),
        in_specs=[rowblk(IN_CH), rowblk(IN_CH),
                  full((IN_CH, HID)), full((1, HID)),
                  full((IN_CH, HID)), full((1, HID)),
                  full((HID, HID)), full((HID, HID)), full((HID, HID))],
        out_specs=[rowblk(HID)] * 4,
        out_shape=[jax.ShapeDtypeStruct((N, HID), jnp.float32)] * 4,
    )(z, pos_enc, W_h, _row(b_h), W_pe, _row(b_pe), WQ, WK, WV)

    out = pl.pallas_call(
        _attn_ffn_kernel,
        grid=(GRID,),
        in_specs=[pl.BlockSpec((N, BLK), lambda i: (0, i)),
                  rowblk(HID),
                  full((N, HID)), full((N, HID)),
                  rowblk(HID),
                  full((HID, HID)), full((1, HID)),
                  full((HID, 2 * HID)), full((1, 2 * HID)),
                  full((2 * HID, HID)), full((1, HID)),
                  full((1, HID)), full((1, HID)),
                  full((1, HID)), full((1, HID))],
        out_specs=rowblk(HID),
        out_shape=jax.ShapeDtypeStruct((N, HID), jnp.float32),
    )(adj, Q, K, V, h, WO, _row(bO), W1, _row(b1), W2, _row(b2),
      _row(bn1_g), _row(bn1_b), _row(bn2_g), _row(bn2_b))
    return out
